# baseline (speedup 1.0000x reference)
import functools

import jax
import jax.numpy as jnp
from jax import lax
from jax.experimental import pallas as pl
from jax.experimental.pallas import tpu as pltpu
from jax.experimental.pallas import tpu_sc as plsc

NC = 2
NS = 16
NW = NC * NS
CHUNK = 128
NBUF = 10
L = 8


def _sc_gather(tot, d, dtype):
    per_w = tot // NW
    n_chunks = per_w // CHUNK
    assert per_w * NW == tot and n_chunks * CHUNK == per_w
    assert n_chunks % NBUF == 0 and n_chunks > NBUF

    mesh = plsc.VectorSubcoreMesh(
        core_axis_name="c", subcore_axis_name="s",
        num_cores=NC, num_subcores=NS)

    @functools.partial(
        pl.kernel,
        out_type=jax.ShapeDtypeStruct((NW, per_w, d), dtype),
        mesh=mesh,
        scratch_types=[
            pltpu.VMEM((n_chunks, CHUNK), jnp.int32),
            pltpu.VMEM((NBUF, CHUNK, d), dtype),
            [pltpu.SemaphoreType.DMA] * NBUF,
            [pltpu.SemaphoreType.DMA] * NBUF,
        ],
        compiler_params=pltpu.CompilerParams(use_tc_tiling_on_sc=False),
    )
    def run(tab_hbm, idx_hbm, out_hbm, idx_v, rows_v, gsem, ssem):
        wid = lax.axis_index("s") * NC + lax.axis_index("c")
        pltpu.sync_copy(idx_hbm.at[wid], idx_v)

        def fire_gather(j, b):
            pltpu.async_copy(tab_hbm.at[idx_v.at[j]], rows_v.at[b], gsem[b])

        def wait_gather(j, b):
            pltpu.make_async_copy(
                tab_hbm.at[idx_v.at[j]], rows_v.at[b], gsem[b]).wait()

        def fire_scatter(j, b):
            pltpu.async_copy(
                rows_v.at[b], out_hbm.at[wid, pl.ds(j * CHUNK, CHUNK)],
                ssem[b])

        def wait_scatter(b):
            pltpu.make_async_copy(
                rows_v.at[b], out_hbm.at[wid, pl.ds(0, CHUNK)],
                ssem[b]).wait()

        for j in range(L):
            fire_gather(j, j % NBUF)

        for j in range(NBUF - L):
            wait_gather(j, j)
            fire_scatter(j, j)
            fire_gather(j + L, (j + L) % NBUF)

        n_steady = n_chunks - NBUF
        @pl.loop(0, n_steady // NBUF)
        def _step(t):
            for u in range(NBUF):
                j = (NBUF - L) + t * NBUF + u
                b = (NBUF - L + u) % NBUF
                bn = (b + L) % NBUF
                wait_gather(j, b)
                fire_scatter(j, b)
                wait_scatter(bn)
                fire_gather(j + L, bn)

        for j in range(n_chunks - L, n_chunks):
            b = j % NBUF
            wait_gather(j, b)
            fire_scatter(j, b)

        for b in range(NBUF):
            wait_scatter(b)

    return run


def kernel(x, table):
    b, h = x.shape
    n, d = table.shape
    tot = b * h
    idx = x.reshape(NW, tot // NW // CHUNK, CHUNK).astype(jnp.int32)
    out = _sc_gather(tot, d, table.dtype)(table, idx)
    return out.reshape(b, h, d)
